# baseline (speedup 1.0000x reference)
import functools

import jax
import jax.numpy as jnp
from jax import lax
from jax.experimental import pallas as pl
from jax.experimental.pallas import tpu as pltpu
from jax.experimental.pallas import tpu_sc as plsc

N_ENT = 10000
N_REL = 8
N_EDGE = 160000
D = 128

NC, NS, L = 2, 16, 16
NW = NC * NS
CH = 128
SUP = 8
NSUP = 5
SPAN = SUP * CH * NSUP
E_PAD = SPAN * NW
ACC_PAD = 10112
ROWS_T = ACC_PAD // NS
TRASH = ACC_PAD - 1
CNT_PAD = 10240
NID_PAD = 10240
SELF_W = NID_PAD // NW

_mesh = plsc.VectorSubcoreMesh(core_axis_name="c", subcore_axis_name="s")


BN = 400


def _ymat_body(x_ref, r_ref, y_ref):
    y_ref[0] = jnp.dot(x_ref[...], r_ref[0], preferred_element_type=jnp.float32)


def _ymat(entity_emb, relation_matrix):
    nb = N_ENT // BN
    return pl.pallas_call(
        _ymat_body,
        grid=(N_REL + 1, nb),
        in_specs=[
            pl.BlockSpec((BN, D), lambda r, j: (j, 0)),
            pl.BlockSpec((1, D, D), lambda r, j: (r, 0, 0)),
        ],
        out_specs=pl.BlockSpec((1, BN, D), lambda r, j: (r, j, 0)),
        out_shape=jax.ShapeDtypeStruct((N_REL + 1, N_ENT, D), jnp.float32),
    )(entity_emb, relation_matrix)


@functools.partial(
    pl.kernel,
    mesh=_mesh,
    out_type=[
        jax.ShapeDtypeStruct((NC, ACC_PAD, D), jnp.float32),
        jax.ShapeDtypeStruct((NC, CNT_PAD), jnp.float32),
    ],
    scratch_types=[
        pltpu.VMEM((SUP, CH), jnp.int32),
        pltpu.VMEM((SUP, CH), jnp.int32),
        pltpu.VMEM((SUP, CH), jnp.int32),
        pltpu.VMEM((SUP, CH), jnp.int32),
        pltpu.VMEM((2, CH, D), jnp.float32),
        pltpu.VMEM((SELF_W // 80, 80), jnp.int32),
        pltpu.VMEM((CH,), jnp.float32),
        pltpu.VMEM((CNT_PAD // NS,), jnp.float32),
        pltpu.VMEM_SHARED((ACC_PAD, D), jnp.float32),
        pltpu.VMEM_SHARED((CNT_PAD,), jnp.float32),
        pltpu.SemaphoreType.DMA,
        pltpu.SemaphoreType.DMA,
        pltpu.SemaphoreType.DMA,
    ],
)
def _scatter(eid2d_hbm, eaflat_hbm, src2d_hbm, dst2d_hbm, nidp_hbm, zflat_hbm,
             pacc_hbm, pcnt_hbm,
             bufE, bufT, bufS, bufD, rows2, selfidx, ones, zflat,
             accS, cntS, semT, semG0, semG1):
    c = lax.axis_index("c")
    s = lax.axis_index("s")
    wid = s * NC + c
    semG = (semG0, semG1)
    lane = lax.iota(jnp.int32, L)

    def zb_body(i, _):
        for j in range(D // L):
            rows2[0, i, pl.ds(j * L, L)] = jnp.zeros((L,), jnp.float32)
        return 0

    lax.fori_loop(0, CH, zb_body, 0)

    def zf_body(k, _):
        zflat[pl.ds(k * L, L)] = jnp.zeros((L,), jnp.float32)
        return 0

    lax.fori_loop(0, (CNT_PAD // NS) // L, zf_body, 0)
    for j in range(CH // L):
        ones[pl.ds(j * L, L)] = jnp.ones((L,), jnp.float32)

    zchunks = [CH] * (ROWS_T // CH) + ([ROWS_T % CH] if ROWS_T % CH else [])
    off = 0
    for n in zchunks:
        pltpu.sync_copy(rows2.at[0, pl.ds(0, n)],
                        accS.at[pl.ds(s * ROWS_T + off, n)])
        off += n
    pltpu.sync_copy(zflat, cntS.at[pl.ds(s * (CNT_PAD // NS), CNT_PAD // NS)])
    plsc.subcore_barrier()

    vbase = c * (NID_PAD // NC) + s * SELF_W
    for j in range(SELF_W // 80):
        vj = vbase + j * 80
        for g in range(80 // L):
            v16 = vj + g * L + lane
            selfidx[j, pl.ds(g * L, L)] = jnp.where(
                v16 < N_ENT, v16, N_ENT + (v16 & 63))
        pltpu.sync_copy(nidp_hbm.at[pl.ds(vj, 80)], bufE.at[0, pl.ds(0, 80)])
        for g in range(80 // L):
            bufT[0, pl.ds(g * L, L)] = (
                bufE[0, pl.ds(g * L, L)] + N_REL * N_ENT)
        pltpu.async_copy(
            zflat_hbm.at[bufT.at[0, pl.ds(0, 80)]],
            rows2.at[0, pl.ds(0, 80)], semG0).wait()
        pltpu.sync_copy(rows2.at[0, pl.ds(0, 80)],
                        accS.at[selfidx.at[j]], add=True)

    def super_body(si, _):
        r0 = wid * (SPAN // CH) + si * SUP
        base_e = r0 * CH
        pltpu.sync_copy(eid2d_hbm.at[pl.ds(r0, SUP)], bufE)
        for k in range(SUP):
            for j in range(CH // L):
                bufE[k, pl.ds(j * L, L)] = bufE[k, pl.ds(j * L, L)] * 2 + 1
        cps = [pltpu.async_copy(eaflat_hbm.at[bufE.at[k]], bufT.at[k], semT)
               for k in range(SUP)]
        for cp in cps:
            cp.wait()
        pltpu.sync_copy(src2d_hbm.at[pl.ds(r0, SUP)], bufS)
        cps = [pltpu.async_copy(nidp_hbm.at[bufS.at[k]], bufE.at[k], semT)
               for k in range(SUP)]
        for cp in cps:
            cp.wait()
        for k in range(SUP):
            for j in range(CH // L):
                bufT[k, pl.ds(j * L, L)] = (
                    bufT[k, pl.ds(j * L, L)] * N_ENT
                    + bufE[k, pl.ds(j * L, L)])
        pltpu.sync_copy(dst2d_hbm.at[pl.ds(r0, SUP)], bufD)
        for k in range(SUP):
            for j in range(CH // L):
                pos = base_e + k * CH + j * L + lane
                trash = N_ENT + (pos & 63)
                bufD[k, pl.ds(j * L, L)] = jnp.where(
                    pos < N_EDGE, bufD[k, pl.ds(j * L, L)], trash)
        cps = [None, None]
        cps[0] = pltpu.async_copy(
            zflat_hbm.at[bufT.at[0]], rows2.at[0], semG[0])
        for k in range(SUP):
            p = k & 1
            if k + 1 < SUP:
                q = (k + 1) & 1
                cps[q] = pltpu.async_copy(
                    zflat_hbm.at[bufT.at[k + 1]], rows2.at[q], semG[q])
            cps[p].wait()
            pltpu.sync_copy(rows2.at[p], accS.at[bufD.at[k]], add=True)
            pltpu.sync_copy(ones, cntS.at[bufD.at[k]], add=True)
        return 0

    lax.fori_loop(0, NSUP, super_body, 0)
    plsc.subcore_barrier()

    off = 0
    for n in zchunks:
        b = s * ROWS_T + off
        pltpu.sync_copy(accS.at[pl.ds(b, n)], rows2.at[0, pl.ds(0, n)])
        pltpu.sync_copy(rows2.at[0, pl.ds(0, n)],
                        pacc_hbm.at[c, pl.ds(b, n)])
        off += n

    cb = s * (CNT_PAD // NS)
    pltpu.sync_copy(cntS.at[pl.ds(cb, CNT_PAD // NS)], zflat)
    pltpu.sync_copy(zflat, pcnt_hbm.at[c, pl.ds(cb, CNT_PAD // NS)])


def _fin_body(p_ref, c_ref, o_ref):
    tot = 1.0 + c_ref[0] + c_ref[1]
    o_ref[...] = (p_ref[0] + p_ref[1]) / tot


def _finish(pacc, pcnt):
    nb = N_ENT // BN
    return pl.pallas_call(
        _fin_body,
        grid=(nb,),
        in_specs=[
            pl.BlockSpec((NC, BN, D), lambda j: (0, j, 0)),
            pl.BlockSpec((NC, BN, 1), lambda j: (0, j, 0)),
        ],
        out_specs=pl.BlockSpec((BN, D), lambda j: (j, 0)),
        out_shape=jax.ShapeDtypeStruct((N_ENT, D), jnp.float32),
    )(pacc, pcnt.reshape(NC, CNT_PAD, 1))


def kernel(edge_attr, n_id, e_id, edge_index, entity_emb, relation_emb,
           relation_matrix):
    del relation_emb
    pad = E_PAD - N_EDGE
    eid2d = jnp.concatenate(
        [e_id, jnp.zeros((pad,), jnp.int32)]).reshape(E_PAD // CH, CH)
    src2d = jnp.concatenate(
        [edge_index[0], jnp.zeros((pad,), jnp.int32)]).reshape(E_PAD // CH, CH)
    dst2d = jnp.concatenate(
        [edge_index[1], jnp.zeros((pad,), jnp.int32)]).reshape(E_PAD // CH, CH)
    nidp = jnp.concatenate([n_id, jnp.zeros((NID_PAD - N_ENT,), jnp.int32)])
    ea_flat = edge_attr.reshape(-1)

    zall = _ymat(entity_emb, relation_matrix)
    zflat = zall.reshape((N_REL + 1) * N_ENT, D)
    pacc, pcnt = _scatter(eid2d, ea_flat, src2d, dst2d, nidp, zflat)
    out = _finish(pacc, pcnt)
    return out, n_id, e_id, edge_index


# speedup vs baseline: 1.3810x; 1.3810x over previous
import functools

import jax
import jax.numpy as jnp
from jax import lax
from jax.experimental import pallas as pl
from jax.experimental.pallas import tpu as pltpu
from jax.experimental.pallas import tpu_sc as plsc

N_ENT = 10000
N_REL = 8
N_EDGE = 160000
D = 128

NC, NS, L = 2, 16, 16
NW = NC * NS
CH = 128
SUP = 8
NSUP0 = 7
NSUP1 = 3
E_PAD = (NSUP0 + NSUP1) * SUP * CH * NS
C0_ROWS = NS * NSUP0 * SUP
ACC_PAD = 10112
ROWS_T = ACC_PAD // NS
TRASH = ACC_PAD - 1
CNT_PAD = 10240
NID_PAD = 10240
SELF_W = NID_PAD // NW

_mesh = plsc.VectorSubcoreMesh(core_axis_name="c", subcore_axis_name="s")


BN = 400


def _ymat_body(x_ref, r_ref, y_ref):
    y_ref[0] = jnp.dot(x_ref[...], r_ref[0], preferred_element_type=jnp.float32)


def _ymat(entity_emb, relation_matrix):
    return pl.pallas_call(
        _ymat_body,
        grid=(N_REL + 1,),
        in_specs=[
            pl.BlockSpec((N_ENT, D), lambda r: (0, 0)),
            pl.BlockSpec((1, D, D), lambda r: (r, 0, 0)),
        ],
        out_specs=pl.BlockSpec((1, N_ENT, D), lambda r: (r, 0, 0)),
        out_shape=jax.ShapeDtypeStruct((N_REL + 1, N_ENT, D), jnp.float32),
    )(entity_emb, relation_matrix)


@functools.partial(
    pl.kernel,
    mesh=_mesh,
    out_type=[
        jax.ShapeDtypeStruct((NC, ACC_PAD, D), jnp.float32),
        jax.ShapeDtypeStruct((NC, CNT_PAD), jnp.float32),
    ],
    scratch_types=[
        pltpu.VMEM((SUP, CH), jnp.int32),
        pltpu.VMEM((SUP, CH), jnp.int32),
        pltpu.VMEM((SUP, CH), jnp.int32),
        pltpu.VMEM((SUP, CH), jnp.int32),
        pltpu.VMEM((2, CH, D), jnp.float32),
        pltpu.VMEM((SELF_W // 80, 80), jnp.int32),
        pltpu.VMEM((CH,), jnp.float32),
        pltpu.VMEM((CNT_PAD // NS,), jnp.float32),
        pltpu.VMEM_SHARED((ACC_PAD, D), jnp.float32),
        pltpu.VMEM_SHARED((CNT_PAD,), jnp.float32),
        pltpu.SemaphoreType.DMA,
        pltpu.SemaphoreType.DMA,
        pltpu.SemaphoreType.DMA,
    ],
)
def _scatter(eid2d_hbm, eaflat_hbm, src2d_hbm, dst2d_hbm, nidp_hbm, zflat_hbm,
             pacc_hbm, pcnt_hbm,
             bufE, bufT, bufS, bufD, rows2, selfidx, ones, zflat,
             accS, cntS, semT, semG0, semG1):
    c = lax.axis_index("c")
    s = lax.axis_index("s")
    wid = s * NC + c
    semG = (semG0, semG1)
    lane = lax.iota(jnp.int32, L)

    def zb_body(i, _):
        for j in range(D // L):
            rows2[0, i, pl.ds(j * L, L)] = jnp.zeros((L,), jnp.float32)
        return 0

    lax.fori_loop(0, CH, zb_body, 0)

    def zf_body(k, _):
        zflat[pl.ds(k * L, L)] = jnp.zeros((L,), jnp.float32)
        return 0

    lax.fori_loop(0, (CNT_PAD // NS) // L, zf_body, 0)
    for j in range(CH // L):
        ones[pl.ds(j * L, L)] = jnp.ones((L,), jnp.float32)

    zchunks = [CH] * (ROWS_T // CH) + ([ROWS_T % CH] if ROWS_T % CH else [])
    off = 0
    for n in zchunks:
        pltpu.sync_copy(rows2.at[0, pl.ds(0, n)],
                        accS.at[pl.ds(s * ROWS_T + off, n)])
        off += n
    pltpu.sync_copy(zflat, cntS.at[pl.ds(s * (CNT_PAD // NS), CNT_PAD // NS)])
    plsc.subcore_barrier()

    vbase = c * (NID_PAD // NC) + s * SELF_W
    for j in range(SELF_W // 80):
        vj = vbase + j * 80
        for g in range(80 // L):
            v16 = vj + g * L + lane
            selfidx[j, pl.ds(g * L, L)] = jnp.where(
                v16 < N_ENT, v16, N_ENT + (v16 & 63))
        pltpu.sync_copy(nidp_hbm.at[pl.ds(vj, 80)], bufE.at[0, pl.ds(0, 80)])
        for g in range(80 // L):
            bufT[0, pl.ds(g * L, L)] = (
                bufE[0, pl.ds(g * L, L)] + N_REL * N_ENT)
        pltpu.async_copy(
            zflat_hbm.at[bufT.at[0, pl.ds(0, 80)]],
            rows2.at[0, pl.ds(0, 80)], semG0).wait()
        pltpu.sync_copy(rows2.at[0, pl.ds(0, 80)],
                        accS.at[selfidx.at[j]], add=True)

    def super_body(si, _):
        r0 = jnp.where(c == 0, s * (NSUP0 * SUP),
                       C0_ROWS + s * (NSUP1 * SUP)) + si * SUP
        base_e = r0 * CH
        pltpu.sync_copy(eid2d_hbm.at[pl.ds(r0, SUP)], bufE)
        for k in range(SUP):
            for j in range(CH // L):
                bufE[k, pl.ds(j * L, L)] = bufE[k, pl.ds(j * L, L)] * 2 + 1
        cps = [pltpu.async_copy(eaflat_hbm.at[bufE.at[k]], bufT.at[k], semT)
               for k in range(SUP)]
        for cp in cps:
            cp.wait()
        pltpu.sync_copy(src2d_hbm.at[pl.ds(r0, SUP)], bufS)
        cps = [pltpu.async_copy(nidp_hbm.at[bufS.at[k]], bufE.at[k], semT)
               for k in range(SUP)]
        for cp in cps:
            cp.wait()
        for k in range(SUP):
            for j in range(CH // L):
                bufT[k, pl.ds(j * L, L)] = (
                    bufT[k, pl.ds(j * L, L)] * N_ENT
                    + bufE[k, pl.ds(j * L, L)])
        pltpu.sync_copy(dst2d_hbm.at[pl.ds(r0, SUP)], bufD)
        for k in range(SUP):
            for j in range(CH // L):
                pos = base_e + k * CH + j * L + lane
                trash = N_ENT + (pos & 63)
                bufD[k, pl.ds(j * L, L)] = jnp.where(
                    pos < N_EDGE, bufD[k, pl.ds(j * L, L)], trash)
        cps = [None, None]
        cps[0] = pltpu.async_copy(
            zflat_hbm.at[bufT.at[0]], rows2.at[0], semG[0])
        for k in range(SUP):
            p = k & 1
            if k + 1 < SUP:
                q = (k + 1) & 1
                cps[q] = pltpu.async_copy(
                    zflat_hbm.at[bufT.at[k + 1]], rows2.at[q], semG[q])
            cps[p].wait()
            pltpu.sync_copy(rows2.at[p], accS.at[bufD.at[k]], add=True)
            pltpu.sync_copy(ones, cntS.at[bufD.at[k]], add=True)
        return 0

    lax.fori_loop(0, jnp.where(c == 0, NSUP0, NSUP1), super_body, 0)
    plsc.subcore_barrier()

    off = 0
    for n in zchunks:
        b = s * ROWS_T + off
        pltpu.sync_copy(accS.at[pl.ds(b, n)], rows2.at[0, pl.ds(0, n)])
        pltpu.sync_copy(rows2.at[0, pl.ds(0, n)],
                        pacc_hbm.at[c, pl.ds(b, n)])
        off += n

    cb = s * (CNT_PAD // NS)
    pltpu.sync_copy(cntS.at[pl.ds(cb, CNT_PAD // NS)], zflat)
    pltpu.sync_copy(zflat, pcnt_hbm.at[c, pl.ds(cb, CNT_PAD // NS)])


def _fin_body(p_ref, c_ref, o_ref):
    tot = 1.0 + c_ref[0] + c_ref[1]
    o_ref[...] = (p_ref[0] + p_ref[1]) / tot


def _finish(pacc, pcnt):
    nb = N_ENT // BN
    return pl.pallas_call(
        _fin_body,
        grid=(nb,),
        in_specs=[
            pl.BlockSpec((NC, BN, D), lambda j: (0, j, 0)),
            pl.BlockSpec((NC, BN, 1), lambda j: (0, j, 0)),
        ],
        out_specs=pl.BlockSpec((BN, D), lambda j: (j, 0)),
        out_shape=jax.ShapeDtypeStruct((N_ENT, D), jnp.float32),
    )(pacc, pcnt.reshape(NC, CNT_PAD, 1))


def kernel(edge_attr, n_id, e_id, edge_index, entity_emb, relation_emb,
           relation_matrix):
    del relation_emb
    pad = E_PAD - N_EDGE
    eid2d = jnp.concatenate(
        [e_id, jnp.zeros((pad,), jnp.int32)]).reshape(E_PAD // CH, CH)
    src2d = jnp.concatenate(
        [edge_index[0], jnp.zeros((pad,), jnp.int32)]).reshape(E_PAD // CH, CH)
    dst2d = jnp.concatenate(
        [edge_index[1], jnp.zeros((pad,), jnp.int32)]).reshape(E_PAD // CH, CH)
    nidp = jnp.concatenate([n_id, jnp.zeros((NID_PAD - N_ENT,), jnp.int32)])
    ea_flat = edge_attr.reshape(-1)

    zall = _ymat(entity_emb, relation_matrix)
    zflat = zall.reshape((N_REL + 1) * N_ENT, D)
    pacc, pcnt = _scatter(eid2d, ea_flat, src2d, dst2d, nidp, zflat)
    out = _finish(pacc, pcnt)
    return out, n_id, e_id, edge_index


# speedup vs baseline: 1.4878x; 1.0774x over previous
import functools

import jax
import jax.numpy as jnp
from jax import lax
from jax.experimental import pallas as pl
from jax.experimental.pallas import tpu as pltpu
from jax.experimental.pallas import tpu_sc as plsc

N_ENT = 10000
N_REL = 8
N_EDGE = 160000
D = 128

NC, NS, L = 2, 16, 16
NW = NC * NS
CH = 128
SUP = 8
NSUP0 = 8
NSUP1 = 2
E_PAD = (NSUP0 + NSUP1) * SUP * CH * NS
C0_ROWS = NS * NSUP0 * SUP
ACC_PAD = 10112
ROWS_T = ACC_PAD // NS
TRASH = ACC_PAD - 1
CNT_PAD = 10240
NID_PAD = 10240
SELF_W = NID_PAD // NW

_mesh = plsc.VectorSubcoreMesh(core_axis_name="c", subcore_axis_name="s")


BN = 400


def _ymat_body(x_ref, r_ref, y_ref):
    y_ref[0] = jnp.dot(x_ref[...], r_ref[0], preferred_element_type=jnp.float32)


def _ymat(entity_emb, relation_matrix):
    return pl.pallas_call(
        _ymat_body,
        grid=(N_REL + 1,),
        in_specs=[
            pl.BlockSpec((N_ENT, D), lambda r: (0, 0)),
            pl.BlockSpec((1, D, D), lambda r: (r, 0, 0)),
        ],
        out_specs=pl.BlockSpec((1, N_ENT, D), lambda r: (r, 0, 0)),
        out_shape=jax.ShapeDtypeStruct((N_REL + 1, N_ENT, D), jnp.float32),
    )(entity_emb, relation_matrix)


@functools.partial(
    pl.kernel,
    mesh=_mesh,
    out_type=[
        jax.ShapeDtypeStruct((NC, ACC_PAD, D), jnp.float32),
        jax.ShapeDtypeStruct((NC, CNT_PAD), jnp.float32),
    ],
    scratch_types=[
        pltpu.VMEM((SUP, CH), jnp.int32),
        pltpu.VMEM((SUP, CH), jnp.int32),
        pltpu.VMEM((SUP, CH), jnp.int32),
        pltpu.VMEM((SUP, CH), jnp.int32),
        pltpu.VMEM((2, CH, D), jnp.float32),
        pltpu.VMEM((SELF_W // 80, 80), jnp.int32),
        pltpu.VMEM((CH,), jnp.float32),
        pltpu.VMEM((CNT_PAD // NS,), jnp.float32),
        pltpu.VMEM_SHARED((ACC_PAD, D), jnp.float32),
        pltpu.VMEM_SHARED((CNT_PAD,), jnp.float32),
        pltpu.SemaphoreType.DMA,
        pltpu.SemaphoreType.DMA,
        pltpu.SemaphoreType.DMA,
    ],
)
def _scatter(eid2d_hbm, eaflat_hbm, src2d_hbm, dst2d_hbm, nidp_hbm, zflat_hbm,
             pacc_hbm, pcnt_hbm,
             bufE, bufT, bufS, bufD, rows2, selfidx, ones, zflat,
             accS, cntS, semT, semG0, semG1):
    c = lax.axis_index("c")
    s = lax.axis_index("s")
    wid = s * NC + c
    semG = (semG0, semG1)
    lane = lax.iota(jnp.int32, L)

    def zb_body(i, _):
        for j in range(D // L):
            rows2[0, i, pl.ds(j * L, L)] = jnp.zeros((L,), jnp.float32)
        return 0

    lax.fori_loop(0, CH, zb_body, 0)

    def zf_body(k, _):
        zflat[pl.ds(k * L, L)] = jnp.zeros((L,), jnp.float32)
        return 0

    lax.fori_loop(0, (CNT_PAD // NS) // L, zf_body, 0)
    for j in range(CH // L):
        ones[pl.ds(j * L, L)] = jnp.ones((L,), jnp.float32)

    zchunks = [CH] * (ROWS_T // CH) + ([ROWS_T % CH] if ROWS_T % CH else [])
    off = 0
    for n in zchunks:
        pltpu.sync_copy(rows2.at[0, pl.ds(0, n)],
                        accS.at[pl.ds(s * ROWS_T + off, n)])
        off += n
    pltpu.sync_copy(zflat, cntS.at[pl.ds(s * (CNT_PAD // NS), CNT_PAD // NS)])
    plsc.subcore_barrier()

    vbase = c * (NID_PAD // NC) + s * SELF_W
    for j in range(SELF_W // 80):
        vj = vbase + j * 80
        for g in range(80 // L):
            v16 = vj + g * L + lane
            selfidx[j, pl.ds(g * L, L)] = jnp.where(
                v16 < N_ENT, v16, N_ENT + (v16 & 63))
        pltpu.sync_copy(nidp_hbm.at[pl.ds(vj, 80)], bufE.at[0, pl.ds(0, 80)])
        for g in range(80 // L):
            bufT[0, pl.ds(g * L, L)] = (
                bufE[0, pl.ds(g * L, L)] + N_REL * N_ENT)
        pltpu.async_copy(
            zflat_hbm.at[bufT.at[0, pl.ds(0, 80)]],
            rows2.at[0, pl.ds(0, 80)], semG0).wait()
        pltpu.sync_copy(rows2.at[0, pl.ds(0, 80)],
                        accS.at[selfidx.at[j]], add=True)

    def super_body(si, _):
        r0 = jnp.where(c == 0, s * (NSUP0 * SUP),
                       C0_ROWS + s * (NSUP1 * SUP)) + si * SUP
        base_e = r0 * CH
        pltpu.sync_copy(eid2d_hbm.at[pl.ds(r0, SUP)], bufE)
        for k in range(SUP):
            for j in range(CH // L):
                bufE[k, pl.ds(j * L, L)] = bufE[k, pl.ds(j * L, L)] * 2 + 1
        cps = [pltpu.async_copy(eaflat_hbm.at[bufE.at[k]], bufT.at[k], semT)
               for k in range(SUP)]
        for cp in cps:
            cp.wait()
        pltpu.sync_copy(src2d_hbm.at[pl.ds(r0, SUP)], bufS)
        cps = [pltpu.async_copy(nidp_hbm.at[bufS.at[k]], bufE.at[k], semT)
               for k in range(SUP)]
        for cp in cps:
            cp.wait()
        for k in range(SUP):
            for j in range(CH // L):
                bufT[k, pl.ds(j * L, L)] = (
                    bufT[k, pl.ds(j * L, L)] * N_ENT
                    + bufE[k, pl.ds(j * L, L)])
        pltpu.sync_copy(dst2d_hbm.at[pl.ds(r0, SUP)], bufD)
        for k in range(SUP):
            for j in range(CH // L):
                pos = base_e + k * CH + j * L + lane
                trash = N_ENT + (pos & 63)
                bufD[k, pl.ds(j * L, L)] = jnp.where(
                    pos < N_EDGE, bufD[k, pl.ds(j * L, L)], trash)
        cps = [None, None]
        cps[0] = pltpu.async_copy(
            zflat_hbm.at[bufT.at[0]], rows2.at[0], semG[0])
        for k in range(SUP):
            p = k & 1
            if k + 1 < SUP:
                q = (k + 1) & 1
                cps[q] = pltpu.async_copy(
                    zflat_hbm.at[bufT.at[k + 1]], rows2.at[q], semG[q])
            cps[p].wait()
            pltpu.sync_copy(rows2.at[p], accS.at[bufD.at[k]], add=True)
            pltpu.sync_copy(ones, cntS.at[bufD.at[k]], add=True)
        return 0

    lax.fori_loop(0, jnp.where(c == 0, NSUP0, NSUP1), super_body, 0)
    plsc.subcore_barrier()

    off = 0
    for n in zchunks:
        b = s * ROWS_T + off
        pltpu.sync_copy(accS.at[pl.ds(b, n)], rows2.at[0, pl.ds(0, n)])
        pltpu.sync_copy(rows2.at[0, pl.ds(0, n)],
                        pacc_hbm.at[c, pl.ds(b, n)])
        off += n

    cb = s * (CNT_PAD // NS)
    pltpu.sync_copy(cntS.at[pl.ds(cb, CNT_PAD // NS)], zflat)
    pltpu.sync_copy(zflat, pcnt_hbm.at[c, pl.ds(cb, CNT_PAD // NS)])


def _fin_body(p_ref, c_ref, o_ref):
    tot = 1.0 + c_ref[0] + c_ref[1]
    o_ref[...] = (p_ref[0] + p_ref[1]) / tot


def _finish(pacc, pcnt):
    nb = N_ENT // BN
    return pl.pallas_call(
        _fin_body,
        grid=(nb,),
        in_specs=[
            pl.BlockSpec((NC, BN, D), lambda j: (0, j, 0)),
            pl.BlockSpec((NC, BN, 1), lambda j: (0, j, 0)),
        ],
        out_specs=pl.BlockSpec((BN, D), lambda j: (j, 0)),
        out_shape=jax.ShapeDtypeStruct((N_ENT, D), jnp.float32),
    )(pacc, pcnt.reshape(NC, CNT_PAD, 1))


def kernel(edge_attr, n_id, e_id, edge_index, entity_emb, relation_emb,
           relation_matrix):
    del relation_emb
    pad = E_PAD - N_EDGE
    eid2d = jnp.concatenate(
        [e_id, jnp.zeros((pad,), jnp.int32)]).reshape(E_PAD // CH, CH)
    src2d = jnp.concatenate(
        [edge_index[0], jnp.zeros((pad,), jnp.int32)]).reshape(E_PAD // CH, CH)
    dst2d = jnp.concatenate(
        [edge_index[1], jnp.zeros((pad,), jnp.int32)]).reshape(E_PAD // CH, CH)
    nidp = jnp.concatenate([n_id, jnp.zeros((NID_PAD - N_ENT,), jnp.int32)])
    ea_flat = edge_attr.reshape(-1)

    zall = _ymat(entity_emb, relation_matrix)
    zflat = zall.reshape((N_REL + 1) * N_ENT, D)
    pacc, pcnt = _scatter(eid2d, ea_flat, src2d, dst2d, nidp, zflat)
    out = _finish(pacc, pcnt)
    return out, n_id, e_id, edge_index


# speedup vs baseline: 1.4903x; 1.0017x over previous
import functools

import jax
import jax.numpy as jnp
from jax import lax
from jax.experimental import pallas as pl
from jax.experimental.pallas import tpu as pltpu
from jax.experimental.pallas import tpu_sc as plsc

N_ENT = 10000
N_REL = 8
N_EDGE = 160000
D = 128

NC, NS, L = 2, 16, 16
NW = NC * NS
CH = 128
SUP = 8
NSUP0 = 9
NSUP1 = 1
E_PAD = (NSUP0 + NSUP1) * SUP * CH * NS
C0_ROWS = NS * NSUP0 * SUP
ACC_PAD = 10112
ROWS_T = ACC_PAD // NS
TRASH = ACC_PAD - 1
CNT_PAD = 10240
NID_PAD = 10240
SELF_W = NID_PAD // NW

_mesh = plsc.VectorSubcoreMesh(core_axis_name="c", subcore_axis_name="s")


BN = 400


def _ymat_body(x_ref, r_ref, y_ref):
    y_ref[0] = jnp.dot(x_ref[...], r_ref[0], preferred_element_type=jnp.float32)


def _ymat(entity_emb, relation_matrix):
    return pl.pallas_call(
        _ymat_body,
        grid=(N_REL + 1,),
        in_specs=[
            pl.BlockSpec((N_ENT, D), lambda r: (0, 0)),
            pl.BlockSpec((1, D, D), lambda r: (r, 0, 0)),
        ],
        out_specs=pl.BlockSpec((1, N_ENT, D), lambda r: (r, 0, 0)),
        out_shape=jax.ShapeDtypeStruct((N_REL + 1, N_ENT, D), jnp.float32),
    )(entity_emb, relation_matrix)


@functools.partial(
    pl.kernel,
    mesh=_mesh,
    out_type=[
        jax.ShapeDtypeStruct((NC, ACC_PAD, D), jnp.float32),
        jax.ShapeDtypeStruct((NC, CNT_PAD), jnp.float32),
    ],
    scratch_types=[
        pltpu.VMEM((SUP, CH), jnp.int32),
        pltpu.VMEM((SUP, CH), jnp.int32),
        pltpu.VMEM((SUP, CH), jnp.int32),
        pltpu.VMEM((SUP, CH), jnp.int32),
        pltpu.VMEM((2, CH, D), jnp.float32),
        pltpu.VMEM((SELF_W // 80, 80), jnp.int32),
        pltpu.VMEM((CH,), jnp.float32),
        pltpu.VMEM((CNT_PAD // NS,), jnp.float32),
        pltpu.VMEM_SHARED((ACC_PAD, D), jnp.float32),
        pltpu.VMEM_SHARED((CNT_PAD,), jnp.float32),
        pltpu.SemaphoreType.DMA,
        pltpu.SemaphoreType.DMA,
        pltpu.SemaphoreType.DMA,
    ],
)
def _scatter(eid2d_hbm, eaflat_hbm, src2d_hbm, dst2d_hbm, nidp_hbm, zflat_hbm,
             pacc_hbm, pcnt_hbm,
             bufE, bufT, bufS, bufD, rows2, selfidx, ones, zflat,
             accS, cntS, semT, semG0, semG1):
    c = lax.axis_index("c")
    s = lax.axis_index("s")
    wid = s * NC + c
    semG = (semG0, semG1)
    lane = lax.iota(jnp.int32, L)

    def zb_body(i, _):
        for j in range(D // L):
            rows2[0, i, pl.ds(j * L, L)] = jnp.zeros((L,), jnp.float32)
        return 0

    lax.fori_loop(0, CH, zb_body, 0)

    def zf_body(k, _):
        zflat[pl.ds(k * L, L)] = jnp.zeros((L,), jnp.float32)
        return 0

    lax.fori_loop(0, (CNT_PAD // NS) // L, zf_body, 0)
    for j in range(CH // L):
        ones[pl.ds(j * L, L)] = jnp.ones((L,), jnp.float32)

    zchunks = [CH] * (ROWS_T // CH) + ([ROWS_T % CH] if ROWS_T % CH else [])
    off = 0
    for n in zchunks:
        pltpu.sync_copy(rows2.at[0, pl.ds(0, n)],
                        accS.at[pl.ds(s * ROWS_T + off, n)])
        off += n
    pltpu.sync_copy(zflat, cntS.at[pl.ds(s * (CNT_PAD // NS), CNT_PAD // NS)])
    plsc.subcore_barrier()

    vbase = c * (NID_PAD // NC) + s * SELF_W
    for j in range(SELF_W // 80):
        vj = vbase + j * 80
        for g in range(80 // L):
            v16 = vj + g * L + lane
            selfidx[j, pl.ds(g * L, L)] = jnp.where(
                v16 < N_ENT, v16, N_ENT + (v16 & 63))
        pltpu.sync_copy(nidp_hbm.at[pl.ds(vj, 80)], bufE.at[0, pl.ds(0, 80)])
        for g in range(80 // L):
            bufT[0, pl.ds(g * L, L)] = (
                bufE[0, pl.ds(g * L, L)] + N_REL * N_ENT)
        pltpu.async_copy(
            zflat_hbm.at[bufT.at[0, pl.ds(0, 80)]],
            rows2.at[0, pl.ds(0, 80)], semG0).wait()
        pltpu.sync_copy(rows2.at[0, pl.ds(0, 80)],
                        accS.at[selfidx.at[j]], add=True)

    def super_body(si, _):
        r0 = jnp.where(c == 0, s * (NSUP0 * SUP),
                       C0_ROWS + s * (NSUP1 * SUP)) + si * SUP
        base_e = r0 * CH
        pltpu.sync_copy(eid2d_hbm.at[pl.ds(r0, SUP)], bufE)
        for k in range(SUP):
            for j in range(CH // L):
                bufE[k, pl.ds(j * L, L)] = bufE[k, pl.ds(j * L, L)] * 2 + 1
        cps = [pltpu.async_copy(eaflat_hbm.at[bufE.at[k]], bufT.at[k], semT)
               for k in range(SUP)]
        for cp in cps:
            cp.wait()
        pltpu.sync_copy(src2d_hbm.at[pl.ds(r0, SUP)], bufS)
        cps = [pltpu.async_copy(nidp_hbm.at[bufS.at[k]], bufE.at[k], semT)
               for k in range(SUP)]
        for cp in cps:
            cp.wait()
        for k in range(SUP):
            for j in range(CH // L):
                bufT[k, pl.ds(j * L, L)] = (
                    bufT[k, pl.ds(j * L, L)] * N_ENT
                    + bufE[k, pl.ds(j * L, L)])
        pltpu.sync_copy(dst2d_hbm.at[pl.ds(r0, SUP)], bufD)
        for k in range(SUP):
            for j in range(CH // L):
                pos = base_e + k * CH + j * L + lane
                trash = N_ENT + (pos & 63)
                bufD[k, pl.ds(j * L, L)] = jnp.where(
                    pos < N_EDGE, bufD[k, pl.ds(j * L, L)], trash)
        cps = [None, None]
        cps[0] = pltpu.async_copy(
            zflat_hbm.at[bufT.at[0]], rows2.at[0], semG[0])
        for k in range(SUP):
            p = k & 1
            if k + 1 < SUP:
                q = (k + 1) & 1
                cps[q] = pltpu.async_copy(
                    zflat_hbm.at[bufT.at[k + 1]], rows2.at[q], semG[q])
            cps[p].wait()
            pltpu.sync_copy(rows2.at[p], accS.at[bufD.at[k]], add=True)
            pltpu.sync_copy(ones, cntS.at[bufD.at[k]], add=True)
        return 0

    lax.fori_loop(0, jnp.where(c == 0, NSUP0, NSUP1), super_body, 0)
    plsc.subcore_barrier()

    off = 0
    for n in zchunks:
        b = s * ROWS_T + off
        pltpu.sync_copy(accS.at[pl.ds(b, n)], rows2.at[0, pl.ds(0, n)])
        pltpu.sync_copy(rows2.at[0, pl.ds(0, n)],
                        pacc_hbm.at[c, pl.ds(b, n)])
        off += n

    cb = s * (CNT_PAD // NS)
    pltpu.sync_copy(cntS.at[pl.ds(cb, CNT_PAD // NS)], zflat)
    pltpu.sync_copy(zflat, pcnt_hbm.at[c, pl.ds(cb, CNT_PAD // NS)])


def _fin_body(p_ref, c_ref, o_ref):
    tot = 1.0 + c_ref[0] + c_ref[1]
    o_ref[...] = (p_ref[0] + p_ref[1]) / tot


def _finish(pacc, pcnt):
    nb = N_ENT // BN
    return pl.pallas_call(
        _fin_body,
        grid=(nb,),
        in_specs=[
            pl.BlockSpec((NC, BN, D), lambda j: (0, j, 0)),
            pl.BlockSpec((NC, BN, 1), lambda j: (0, j, 0)),
        ],
        out_specs=pl.BlockSpec((BN, D), lambda j: (j, 0)),
        out_shape=jax.ShapeDtypeStruct((N_ENT, D), jnp.float32),
    )(pacc, pcnt.reshape(NC, CNT_PAD, 1))


def kernel(edge_attr, n_id, e_id, edge_index, entity_emb, relation_emb,
           relation_matrix):
    del relation_emb
    pad = E_PAD - N_EDGE
    eid2d = jnp.concatenate(
        [e_id, jnp.zeros((pad,), jnp.int32)]).reshape(E_PAD // CH, CH)
    src2d = jnp.concatenate(
        [edge_index[0], jnp.zeros((pad,), jnp.int32)]).reshape(E_PAD // CH, CH)
    dst2d = jnp.concatenate(
        [edge_index[1], jnp.zeros((pad,), jnp.int32)]).reshape(E_PAD // CH, CH)
    nidp = jnp.concatenate([n_id, jnp.zeros((NID_PAD - N_ENT,), jnp.int32)])
    ea_flat = edge_attr.reshape(-1)

    zall = _ymat(entity_emb, relation_matrix)
    zflat = zall.reshape((N_REL + 1) * N_ENT, D)
    pacc, pcnt = _scatter(eid2d, ea_flat, src2d, dst2d, nidp, zflat)
    out = _finish(pacc, pcnt)
    return out, n_id, e_id, edge_index
